# Initial kernel scaffold; baseline (speedup 1.0000x reference)
import functools
import jax
import jax.numpy as jnp
from jax import lax
from jax.experimental import pallas as pl
from jax.experimental.pallas import tpu as pltpu
from jax.experimental.pallas import tpu_sc as plsc

VOCAB_N = 1_000_000
BATCH_N = 16384
HIST_N = 200
TOTAL_ITEMS = BATCH_N * HIST_N

NUM_CORES = 2
NUM_SUBCORES = 16
NW = NUM_CORES * NUM_SUBCORES
ITEMS_PER_W = TOTAL_ITEMS // NW
CLIENTS_PER_W = BATCH_N // NW

CHUNK = 4096
N_CHUNKS = ITEMS_PER_W // CHUNK

STAGE_CHUNK = VOCAB_N // 8

_mesh = plsc.VectorSubcoreMesh(core_axis_name="c", subcore_axis_name="s")


@functools.partial(
    pl.kernel,
    out_type=(
        jax.ShapeDtypeStruct((BATCH_N,), jnp.float32),
        jax.ShapeDtypeStruct((TOTAL_ITEMS,), jnp.float32),
    ),
    mesh=_mesh,
    scratch_types=[
        pltpu.VMEM_SHARED((VOCAB_N,), jnp.float32),
        pltpu.VMEM_SHARED((VOCAB_N,), jnp.float32),
        pltpu.VMEM((CHUNK,), jnp.int32),
        pltpu.VMEM((CHUNK,), jnp.float32),
        pltpu.VMEM((CLIENTS_PER_W,), jnp.int32),
        pltpu.VMEM((CLIENTS_PER_W,), jnp.float32),
    ],
)
def _gather_kernel(
    item_tab_hbm,
    client_tab_hbm,
    client_ids_hbm,
    item_ids_hbm,
    out_client_hbm,
    out_items_hbm,
    item_sp,
    client_sp,
    idx_v,
    val_v,
    cidx_v,
    cval_v,
):
    c = lax.axis_index("c")
    s = lax.axis_index("s")
    wid = s * NUM_CORES + c

    @pl.when(s < 8)
    def _stage_item():
        off = s * STAGE_CHUNK
        pltpu.sync_copy(
            item_tab_hbm.at[pl.ds(off, STAGE_CHUNK)],
            item_sp.at[pl.ds(off, STAGE_CHUNK)],
        )

    @pl.when(s >= 8)
    def _stage_client():
        off = (s - 8) * STAGE_CHUNK
        pltpu.sync_copy(
            client_tab_hbm.at[pl.ds(off, STAGE_CHUNK)],
            client_sp.at[pl.ds(off, STAGE_CHUNK)],
        )

    plsc.subcore_barrier()

    cbase = wid * CLIENTS_PER_W
    pltpu.sync_copy(client_ids_hbm.at[pl.ds(cbase, CLIENTS_PER_W)], cidx_v)
    pltpu.sync_copy(client_sp.at[cidx_v], cval_v)
    pltpu.sync_copy(cval_v, out_client_hbm.at[pl.ds(cbase, CLIENTS_PER_W)])

    base = wid * ITEMS_PER_W

    @pl.loop(0, N_CHUNKS)
    def _chunk(i):
        off = base + i * CHUNK
        pltpu.sync_copy(item_ids_hbm.at[pl.ds(off, CHUNK)], idx_v)
        pltpu.sync_copy(item_sp.at[idx_v], val_v)
        pltpu.sync_copy(val_v, out_items_hbm.at[pl.ds(off, CHUNK)])


def kernel(item_id2graph_id, client_id2graph_id, client_ids, item_ids):
    flat_items = item_ids.reshape(-1)
    out_client, out_items = _gather_kernel(
        item_id2graph_id, client_id2graph_id, client_ids, flat_items
    )
    return (out_client, out_items.reshape(BATCH_N, HIST_N))


# speedup vs baseline: 189.4287x; 189.4287x over previous
import functools
import jax
import jax.numpy as jnp
from jax import lax
from jax.experimental import pallas as pl
from jax.experimental.pallas import tpu as pltpu
from jax.experimental.pallas import tpu_sc as plsc

VOCAB_N = 1_000_000
BATCH_N = 16384
HIST_N = 200
TOTAL_ITEMS = BATCH_N * HIST_N

NUM_CORES = 2
NUM_SUBCORES = 16
NW = NUM_CORES * NUM_SUBCORES
ITEMS_PER_W = TOTAL_ITEMS // NW
CLIENTS_PER_W = BATCH_N // NW

CHUNK = 4096
N_CHUNKS = ITEMS_PER_W // CHUNK

STAGE_PER_SUB = 62_496
STAGE_CHUNK = 15_624
N_STAGE = STAGE_PER_SUB // STAGE_CHUNK
STAGE_TAIL = VOCAB_N - NUM_SUBCORES * STAGE_PER_SUB

_mesh = plsc.VectorSubcoreMesh(core_axis_name="c", subcore_axis_name="s")


@functools.partial(
    pl.kernel,
    out_type=(
        jax.ShapeDtypeStruct((BATCH_N,), jnp.float32),
        jax.ShapeDtypeStruct((TOTAL_ITEMS,), jnp.float32),
    ),
    mesh=_mesh,
    scratch_types=[
        pltpu.VMEM_SHARED((VOCAB_N,), jnp.float32),
        pltpu.VMEM((CHUNK,), jnp.int32),
        pltpu.VMEM((CHUNK,), jnp.float32),
        pltpu.VMEM((CLIENTS_PER_W,), jnp.int32),
        pltpu.VMEM((CLIENTS_PER_W,), jnp.float32),
        pltpu.VMEM((STAGE_CHUNK,), jnp.float32),
    ],
)
def _gather_kernel(
    item_tab_hbm,
    client_tab_hbm,
    client_ids_hbm,
    item_ids_hbm,
    out_client_hbm,
    out_items_hbm,
    item_sp,
    idx_v,
    val_v,
    cidx_v,
    cval_v,
    stage_v,
):
    c = lax.axis_index("c")
    s = lax.axis_index("s")
    wid = s * NUM_CORES + c

    base_off = s * STAGE_PER_SUB

    @pl.loop(0, N_STAGE)
    def _st(j):
        off = base_off + j * STAGE_CHUNK
        pltpu.sync_copy(item_tab_hbm.at[pl.ds(off, STAGE_CHUNK)], stage_v)
        pltpu.sync_copy(stage_v, item_sp.at[pl.ds(off, STAGE_CHUNK)])

    @pl.when(s == 0)
    def _st_tail():
        toff = NUM_SUBCORES * STAGE_PER_SUB
        pltpu.sync_copy(
            item_tab_hbm.at[pl.ds(toff, STAGE_TAIL)],
            stage_v.at[pl.ds(0, STAGE_TAIL)],
        )
        pltpu.sync_copy(
            stage_v.at[pl.ds(0, STAGE_TAIL)],
            item_sp.at[pl.ds(toff, STAGE_TAIL)],
        )

    cbase = wid * CLIENTS_PER_W
    pltpu.sync_copy(client_ids_hbm.at[pl.ds(cbase, CLIENTS_PER_W)], cidx_v)
    pltpu.sync_copy(client_tab_hbm.at[cidx_v], cval_v)
    pltpu.sync_copy(cval_v, out_client_hbm.at[pl.ds(cbase, CLIENTS_PER_W)])

    plsc.subcore_barrier()

    base = wid * ITEMS_PER_W

    @pl.loop(0, N_CHUNKS)
    def _chunk(i):
        off = base + i * CHUNK
        pltpu.sync_copy(item_ids_hbm.at[pl.ds(off, CHUNK)], idx_v)
        pltpu.sync_copy(item_sp.at[idx_v], val_v)
        pltpu.sync_copy(val_v, out_items_hbm.at[pl.ds(off, CHUNK)])


def kernel(item_id2graph_id, client_id2graph_id, client_ids, item_ids):
    flat_items = item_ids.reshape(-1)
    out_client, out_items = _gather_kernel(
        item_id2graph_id, client_id2graph_id, client_ids, flat_items
    )
    return (out_client, out_items.reshape(BATCH_N, HIST_N))


# speedup vs baseline: 224.3880x; 1.1846x over previous
import functools
import jax
import jax.numpy as jnp
from jax import lax
from jax.experimental import pallas as pl
from jax.experimental.pallas import tpu as pltpu
from jax.experimental.pallas import tpu_sc as plsc

VOCAB_N = 1_000_000
BATCH_N = 16384
HIST_N = 200
TOTAL_ITEMS = BATCH_N * HIST_N

NUM_CORES = 2
NUM_SUBCORES = 16
NW = NUM_CORES * NUM_SUBCORES
ITEMS_PER_W = TOTAL_ITEMS // NW
CLIENTS_PER_W = BATCH_N // NW

CHUNK = 12_800
N_CHUNKS = ITEMS_PER_W // CHUNK

STAGE_PER_SUB = 62_496
STAGE_CHUNK = 15_624
N_STAGE = STAGE_PER_SUB // STAGE_CHUNK
STAGE_TAIL = VOCAB_N - NUM_SUBCORES * STAGE_PER_SUB

_mesh = plsc.VectorSubcoreMesh(core_axis_name="c", subcore_axis_name="s")


@functools.partial(
    pl.kernel,
    out_type=(
        jax.ShapeDtypeStruct((BATCH_N,), jnp.float32),
        jax.ShapeDtypeStruct((TOTAL_ITEMS,), jnp.float32),
    ),
    mesh=_mesh,
    scratch_types=[
        pltpu.VMEM_SHARED((VOCAB_N,), jnp.float32),
        pltpu.VMEM((CHUNK,), jnp.int32),
        pltpu.VMEM((CHUNK,), jnp.int32),
        pltpu.VMEM((CHUNK,), jnp.float32),
        pltpu.VMEM((CHUNK,), jnp.float32),
        pltpu.VMEM((CLIENTS_PER_W,), jnp.int32),
        pltpu.VMEM((CLIENTS_PER_W,), jnp.float32),
        pltpu.VMEM((STAGE_CHUNK,), jnp.float32),
        pltpu.SemaphoreType.DMA,
        pltpu.SemaphoreType.DMA,
        pltpu.SemaphoreType.DMA,
        pltpu.SemaphoreType.DMA,
    ],
)
def _gather_kernel(
    item_tab_hbm,
    client_tab_hbm,
    client_ids_hbm,
    item_ids_hbm,
    out_client_hbm,
    out_items_hbm,
    item_sp,
    idx0_v,
    idx1_v,
    val0_v,
    val1_v,
    cidx_v,
    cval_v,
    stage_v,
    si0,
    si1,
    so0,
    so1,
):
    c = lax.axis_index("c")
    s = lax.axis_index("s")
    wid = s * NUM_CORES + c

    base_off = s * STAGE_PER_SUB

    @pl.loop(0, N_STAGE)
    def _st(j):
        off = base_off + j * STAGE_CHUNK
        pltpu.sync_copy(item_tab_hbm.at[pl.ds(off, STAGE_CHUNK)], stage_v)
        pltpu.sync_copy(stage_v, item_sp.at[pl.ds(off, STAGE_CHUNK)])

    @pl.when(s == 0)
    def _st_tail():
        toff = NUM_SUBCORES * STAGE_PER_SUB
        pltpu.sync_copy(
            item_tab_hbm.at[pl.ds(toff, STAGE_TAIL)],
            stage_v.at[pl.ds(0, STAGE_TAIL)],
        )
        pltpu.sync_copy(
            stage_v.at[pl.ds(0, STAGE_TAIL)],
            item_sp.at[pl.ds(toff, STAGE_TAIL)],
        )

    cbase = wid * CLIENTS_PER_W
    pltpu.sync_copy(client_ids_hbm.at[pl.ds(cbase, CLIENTS_PER_W)], cidx_v)
    pltpu.sync_copy(client_tab_hbm.at[cidx_v], cval_v)
    pltpu.sync_copy(cval_v, out_client_hbm.at[pl.ds(cbase, CLIENTS_PER_W)])

    plsc.subcore_barrier()

    base = wid * ITEMS_PER_W

    pltpu.async_copy(item_ids_hbm.at[pl.ds(base, CHUNK)], idx0_v, si0)
    pltpu.async_copy(item_ids_hbm.at[pl.ds(base + CHUNK, CHUNK)], idx1_v, si1)

    @pl.loop(0, N_CHUNKS, step=2)
    def _chunk(i):
        off0 = base + i * CHUNK
        off1 = off0 + CHUNK

        pltpu.make_async_copy(
            item_ids_hbm.at[pl.ds(off0, CHUNK)], idx0_v, si0
        ).wait()

        @pl.when(i > 0)
        def _wait_out0():
            pltpu.make_async_copy(
                val0_v, out_items_hbm.at[pl.ds(off0 - 2 * CHUNK, CHUNK)], so0
            ).wait()

        pltpu.sync_copy(item_sp.at[idx0_v], val0_v)
        pltpu.async_copy(val0_v, out_items_hbm.at[pl.ds(off0, CHUNK)], so0)

        @pl.when(i + 2 < N_CHUNKS)
        def _prefetch0():
            pltpu.async_copy(
                item_ids_hbm.at[pl.ds(off0 + 2 * CHUNK, CHUNK)], idx0_v, si0
            )

        pltpu.make_async_copy(
            item_ids_hbm.at[pl.ds(off1, CHUNK)], idx1_v, si1
        ).wait()

        @pl.when(i > 0)
        def _wait_out1():
            pltpu.make_async_copy(
                val1_v, out_items_hbm.at[pl.ds(off1 - 2 * CHUNK, CHUNK)], so1
            ).wait()

        pltpu.sync_copy(item_sp.at[idx1_v], val1_v)
        pltpu.async_copy(val1_v, out_items_hbm.at[pl.ds(off1, CHUNK)], so1)

        @pl.when(i + 2 < N_CHUNKS)
        def _prefetch1():
            pltpu.async_copy(
                item_ids_hbm.at[pl.ds(off1 + 2 * CHUNK, CHUNK)], idx1_v, si1
            )

    pltpu.make_async_copy(
        val0_v, out_items_hbm.at[pl.ds(base + (N_CHUNKS - 2) * CHUNK, CHUNK)], so0
    ).wait()
    pltpu.make_async_copy(
        val1_v, out_items_hbm.at[pl.ds(base + (N_CHUNKS - 1) * CHUNK, CHUNK)], so1
    ).wait()


def kernel(item_id2graph_id, client_id2graph_id, client_ids, item_ids):
    flat_items = item_ids.reshape(-1)
    out_client, out_items = _gather_kernel(
        item_id2graph_id, client_id2graph_id, client_ids, flat_items
    )
    return (out_client, out_items.reshape(BATCH_N, HIST_N))
